# Initial kernel scaffold; baseline (speedup 1.0000x reference)
import jax
import jax.numpy as jnp
from jax.experimental import pallas as pl


def kernel(x, edge_index, h1, h2, h3, W1, b1, W2, b2, Wih1, Whh1, bih1, bhh1, Wih2, Whh2, bih2, bhh2, Wih3, Whh3, bih3, bhh3):
    raise NotImplementedError("write your pallas kernel here")


# speedup vs baseline: 10.7932x; 10.7932x over previous
import functools

import jax
import jax.numpy as jnp
from jax import lax
from jax.experimental import pallas as pl
from jax.experimental.pallas import tpu as pltpu
from jax.experimental.pallas import tpu_sc as plsc

NC = 2
NS = 16
NW = NC * NS
CHUNK = 128


def _sc_degree(nnodes, npad, epad):
    stripe = npad // NS
    ndrain = stripe // CHUNK
    nchunks = epad // (NW * CHUNK)
    mesh = plsc.VectorSubcoreMesh(core_axis_name="c", subcore_axis_name="s")

    @functools.partial(
        pl.kernel,
        out_type=jax.ShapeDtypeStruct((NC, npad, 16), jnp.float32),
        mesh=mesh,
        scratch_types=[
            pltpu.VMEM_SHARED((npad, 16), jnp.float32),
            pltpu.VMEM((CHUNK, 16), jnp.float32),
            pltpu.VMEM((CHUNK, 16), jnp.float32),
            pltpu.VMEM((CHUNK,), jnp.int32),
        ],
    )
    def deg_kernel(dst_hbm, out_hbm, acc_sh, ones_v, tmp_v, idx_v):
        cid = lax.axis_index("c")
        sid = lax.axis_index("s")
        wid = sid * NC + cid

        def init_bufs(i, _):
            ones_v[i, :] = jnp.ones((16,), jnp.float32)
            tmp_v[i, :] = jnp.zeros((16,), jnp.float32)
            return 0
        lax.fori_loop(0, CHUNK, init_bufs, 0)
        for k in range(ndrain):
            pltpu.sync_copy(tmp_v, acc_sh.at[pl.ds(sid * stripe + k * CHUNK, CHUNK)])
        plsc.subcore_barrier()

        def body(j, _):
            base = (wid * nchunks + j) * CHUNK
            pltpu.sync_copy(dst_hbm.at[pl.ds(base, CHUNK)], idx_v)
            pltpu.sync_copy(ones_v, acc_sh.at[idx_v], add=True)
            return 0
        lax.fori_loop(0, nchunks, body, 0)
        plsc.subcore_barrier()

        for k in range(ndrain):
            pltpu.sync_copy(acc_sh.at[pl.ds(sid * stripe + k * CHUNK, CHUNK)], tmp_v)
            pltpu.sync_copy(tmp_v, out_hbm.at[cid, pl.ds(sid * stripe + k * CHUNK, CHUNK)])

    return deg_kernel


def _sc_segment_sum(nnodes, npad, epad, width):
    stripe = npad // NS
    ndrain = stripe // CHUNK
    nchunks = epad // (NW * CHUNK)
    mesh = plsc.VectorSubcoreMesh(core_axis_name="c", subcore_axis_name="s")

    @functools.partial(
        pl.kernel,
        out_type=jax.ShapeDtypeStruct((NC, npad, width), jnp.float32),
        mesh=mesh,
        scratch_types=[
            pltpu.VMEM_SHARED((npad, width), jnp.float32),
            pltpu.VMEM((CHUNK, width), jnp.float32),
            pltpu.VMEM((CHUNK, width), jnp.float32),
            pltpu.VMEM((CHUNK,), jnp.int32),
            pltpu.VMEM((CHUNK,), jnp.int32),
            pltpu.SemaphoreType.DMA,
        ],
    )
    def seg_kernel(y_hbm, src_hbm, dst_hbm, out_hbm,
                   acc_sh, rows_v, tmp_v, sidx_v, didx_v, sem):
        cid = lax.axis_index("c")
        sid = lax.axis_index("s")
        wid = sid * NC + cid

        wlanes = width // 16

        def init_zero(i, _):
            for k in range(wlanes):
                tmp_v[i, pl.ds(k * 16, 16)] = jnp.zeros((16,), jnp.float32)
            return 0
        lax.fori_loop(0, CHUNK, init_zero, 0)
        for k in range(ndrain):
            pltpu.sync_copy(tmp_v, acc_sh.at[pl.ds(sid * stripe + k * CHUNK, CHUNK)])
        plsc.subcore_barrier()

        def body(j, _):
            base = (wid * nchunks + j) * CHUNK
            pltpu.sync_copy(src_hbm.at[pl.ds(base, CHUNK)], sidx_v)
            pltpu.sync_copy(dst_hbm.at[pl.ds(base, CHUNK)], didx_v)
            pltpu.async_copy(y_hbm.at[sidx_v], rows_v, sem).wait()
            pltpu.sync_copy(rows_v, acc_sh.at[didx_v], add=True)
            return 0
        lax.fori_loop(0, nchunks, body, 0)
        plsc.subcore_barrier()

        for k in range(ndrain):
            pltpu.sync_copy(acc_sh.at[pl.ds(sid * stripe + k * CHUNK, CHUNK)], tmp_v)
            pltpu.sync_copy(tmp_v, out_hbm.at[cid, pl.ds(sid * stripe + k * CHUNK, CHUNK)])

    return seg_kernel


def _dis_from(deg_ref):
    degv = deg_ref[0] + deg_ref[1]
    deg = degv[:, 0:1] + 1.0
    return lax.rsqrt(deg)


def _tc_b1(deg_ref, x_ref, w_ref, y_ref):
    dis = _dis_from(deg_ref)
    xw = jnp.dot(x_ref[...], w_ref[...], preferred_element_type=jnp.float32)
    y_ref[...] = xw * dis


def _tc_b2(deg_ref, acc_ref, y1_ref, b1_ref, w2_ref, y2_ref):
    dis = _dis_from(deg_ref)
    z = (acc_ref[0] + acc_ref[1] + y1_ref[...]) * dis + b1_ref[...]
    xo = jnp.maximum(z, 0.0)
    y2_ref[...] = jnp.dot(xo, w2_ref[...],
                          preferred_element_type=jnp.float32) * dis


def _gru(x, h, wih_t, whh_t, bi, bh, hdim):
    gi = jnp.dot(x, wih_t, preferred_element_type=jnp.float32) + bi
    gh = jnp.dot(h, whh_t, preferred_element_type=jnp.float32) + bh
    ir, iz, inn = (gi[:, :hdim], gi[:, hdim:2 * hdim], gi[:, 2 * hdim:])
    hr, hz, hn = (gh[:, :hdim], gh[:, hdim:2 * hdim], gh[:, 2 * hdim:])
    r = jax.nn.sigmoid(ir + hr)
    z = jax.nn.sigmoid(iz + hz)
    n = jnp.tanh(inn + r * hn)
    return (1.0 - z) * n + z * h


def _tc_b3(hdim, deg_ref, acc_ref, y2_ref, b2_ref, h1_ref, h2_ref, h3_ref,
           wih1_ref, whh1_ref, bih1_ref, bhh1_ref,
           wih2_ref, whh2_ref, bih2_ref, bhh2_ref,
           wih3_ref, whh3_ref, bih3_ref, bhh3_ref,
           o1_ref, o2_ref, o3_ref):
    dis = _dis_from(deg_ref)
    z = (acc_ref[0] + acc_ref[1] + y2_ref[...]) * dis + b2_ref[...]
    xo = jnp.maximum(z, 0.0)
    o1 = _gru(xo, h1_ref[...], wih1_ref[...], whh1_ref[...],
              bih1_ref[...], bhh1_ref[...], hdim)
    o2 = _gru(o1, h2_ref[...], wih2_ref[...], whh2_ref[...],
              bih2_ref[...], bhh2_ref[...], hdim)
    o3 = _gru(o2, h3_ref[...], wih3_ref[...], whh3_ref[...],
              bih3_ref[...], bhh3_ref[...], hdim)
    o1_ref[...] = o1
    o2_ref[...] = o2
    o3_ref[...] = o3


def kernel(x, edge_index, h1, h2, h3, W1, b1, W2, b2,
           Wih1, Whh1, bih1, bhh1,
           Wih2, Whh2, bih2, bhh2,
           Wih3, Whh3, bih3, bhh3):
    n, d = x.shape
    hdim = h1.shape[1]
    e = edge_index.shape[1]
    stripe = ((n + 1 + NS * CHUNK - 1) // (NS * CHUNK)) * CHUNK
    npad = NS * stripe
    nchunks = -(-e // (NW * CHUNK))
    epad = nchunks * NW * CHUNK
    padn = epad - e

    src = edge_index[0]
    dst = edge_index[1]
    if padn:
        src = jnp.concatenate([src, jnp.zeros((padn,), jnp.int32)])
        dst = jnp.concatenate([dst, jnp.full((padn,), n, jnp.int32)])

    r = n // 10
    grid = n // r

    deg_p = _sc_degree(n, npad, epad)(dst)[:, :n, :]

    y1 = pl.pallas_call(
        _tc_b1,
        grid=(grid,),
        in_specs=[
            pl.BlockSpec((NC, r, 16), lambda i: (0, i, 0)),
            pl.BlockSpec((r, d), lambda i: (i, 0)),
            pl.BlockSpec((d, hdim), lambda i: (0, 0)),
        ],
        out_specs=pl.BlockSpec((r, hdim), lambda i: (i, 0)),
        out_shape=jax.ShapeDtypeStruct((n, hdim), jnp.float32),
    )(deg_p, x, W1)

    seg = _sc_segment_sum(n, npad, epad, hdim)
    acc1 = seg(y1, src, dst)[:, :n, :]

    y2 = pl.pallas_call(
        _tc_b2,
        grid=(grid,),
        in_specs=[
            pl.BlockSpec((NC, r, 16), lambda i: (0, i, 0)),
            pl.BlockSpec((NC, r, hdim), lambda i: (0, i, 0)),
            pl.BlockSpec((r, hdim), lambda i: (i, 0)),
            pl.BlockSpec((1, hdim), lambda i: (0, 0)),
            pl.BlockSpec((hdim, hdim), lambda i: (0, 0)),
        ],
        out_specs=pl.BlockSpec((r, hdim), lambda i: (i, 0)),
        out_shape=jax.ShapeDtypeStruct((n, hdim), jnp.float32),
    )(deg_p, acc1, y1, b1.reshape(1, hdim), W2)

    acc2 = seg(y2, src, dst)[:, :n, :]

    wspec = pl.BlockSpec((hdim, 3 * hdim), lambda i: (0, 0))
    bspec = pl.BlockSpec((1, 3 * hdim), lambda i: (0, 0))
    hspec = pl.BlockSpec((r, hdim), lambda i: (i, 0))
    o1, o2, o3 = pl.pallas_call(
        functools.partial(_tc_b3, hdim),
        grid=(grid,),
        in_specs=[
            pl.BlockSpec((NC, r, 16), lambda i: (0, i, 0)),
            pl.BlockSpec((NC, r, hdim), lambda i: (0, i, 0)),
            hspec,
            pl.BlockSpec((1, hdim), lambda i: (0, 0)),
            hspec, hspec, hspec,
            wspec, wspec, bspec, bspec,
            wspec, wspec, bspec, bspec,
            wspec, wspec, bspec, bspec,
        ],
        out_specs=[hspec, hspec, hspec],
        out_shape=[jax.ShapeDtypeStruct((n, hdim), jnp.float32)] * 3,
    )(deg_p, acc2, y2, b2.reshape(1, hdim), h1, h2, h3,
      Wih1.T, Whh1.T, bih1.reshape(1, -1), bhh1.reshape(1, -1),
      Wih2.T, Whh2.T, bih2.reshape(1, -1), bhh2.reshape(1, -1),
      Wih3.T, Whh3.T, bih3.reshape(1, -1), bhh3.reshape(1, -1))

    return (o1, o2, o3)
